# baseline (speedup 1.0000x reference)
import functools

import jax
import jax.numpy as jnp
from jax import lax
from jax.experimental import pallas as pl
from jax.experimental.pallas import tpu as pltpu
from jax.experimental.pallas import tpu_sc as plsc

N = 10000
E = 320000
H = 128

NC = 2
NS = 16
NW = NC * NS
L = 16
NG = H // L

NPAD = 10240
NHALF = NPAD // 2

EH = E // 2

EC = 128
ECHUNKS = EH // EC
EC_PER_W = -(-ECHUNKS // NW)

CH = 800
SCHUNKS = (EH // 2) // CH

CH3 = 2000
DCHUNKS3 = E // CH3

NODE_BLK = 2000
NODE_BLK2 = 2048
EDGE_BLK = 2000

_MESH = plsc.VectorSubcoreMesh(core_axis_name="c", subcore_axis_name="s")
NEG = -3.0e38


def _wid():
    return lax.axis_index("s") * NC + lax.axis_index("c")


def _uv_kernel(x_ref, wd_ref, ws_ref, b_ref, u_ref, v_ref):
    x = x_ref[...]
    u_ref[...] = jnp.dot(x, wd_ref[...], preferred_element_type=jnp.float32) + b_ref[...]
    v_ref[...] = jnp.dot(x, ws_ref[...], preferred_element_type=jnp.float32)


def _node_uv(x, wd, ws, b):
    n = x.shape[0]
    blk = NODE_BLK if n == N else NODE_BLK2
    return pl.pallas_call(
        _uv_kernel,
        grid=(n // blk,),
        in_specs=[
            pl.BlockSpec((blk, x.shape[1]), lambda i: (i, 0)),
            pl.BlockSpec((x.shape[1], H), lambda i: (0, 0)),
            pl.BlockSpec((x.shape[1], H), lambda i: (0, 0)),
            pl.BlockSpec((1, H), lambda i: (0, 0)),
        ],
        out_specs=[
            pl.BlockSpec((blk, H), lambda i: (i, 0)),
            pl.BlockSpec((blk, H), lambda i: (i, 0)),
        ],
        out_shape=[
            jax.ShapeDtypeStruct((n, H), jnp.float32),
            jax.ShapeDtypeStruct((n, H), jnp.float32),
        ],
    )(x, wd, ws, b)


def _uv_blocked_kernel(pa_ref, pb_ref, wd_ref, ws_ref, b_ref, u_ref, v_ref):
    x = jnp.maximum(jnp.maximum(pa_ref[0], pa_ref[1]),
                    jnp.maximum(pb_ref[0], pb_ref[1]))
    x = jnp.maximum(x, 0.0)
    u_ref[...] = jnp.dot(x, wd_ref[...], preferred_element_type=jnp.float32) + b_ref[...]
    v_ref[...] = jnp.dot(x, ws_ref[...], preferred_element_type=jnp.float32)


def _node_uv_blocked(pa, pb, wd, ws, b):
    blk = NODE_BLK2
    return pl.pallas_call(
        _uv_blocked_kernel,
        grid=(NPAD // blk,),
        in_specs=[
            pl.BlockSpec((2, blk, H), lambda i: (0, i, 0)),
            pl.BlockSpec((2, blk, H), lambda i: (0, i, 0)),
            pl.BlockSpec((H, H), lambda i: (0, 0)),
            pl.BlockSpec((H, H), lambda i: (0, 0)),
            pl.BlockSpec((1, H), lambda i: (0, 0)),
        ],
        out_specs=[
            pl.BlockSpec((blk, H), lambda i: (i, 0)),
            pl.BlockSpec((blk, H), lambda i: (i, 0)),
        ],
        out_shape=[
            jax.ShapeDtypeStruct((NPAD, H), jnp.float32),
            jax.ShapeDtypeStruct((NPAD, H), jnp.float32),
        ],
    )(pa, pb, wd, ws, b)


def _mm_kernel(h_ref, w_ref, b_ref, m_ref):
    m_ref[...] = jnp.dot(h_ref[...], w_ref[...], preferred_element_type=jnp.float32) + b_ref[...]


def _edge_mm(h, w2t, b2):
    return pl.pallas_call(
        _mm_kernel,
        grid=(EH // EDGE_BLK,),
        in_specs=[
            pl.BlockSpec((EDGE_BLK, H), lambda i: (i, 0)),
            pl.BlockSpec((H, H), lambda i: (0, 0)),
            pl.BlockSpec((1, H), lambda i: (0, 0)),
        ],
        out_specs=pl.BlockSpec((EDGE_BLK, H), lambda i: (i, 0)),
        out_shape=jax.ShapeDtypeStruct((EH, H), jnp.float32),
    )(h, w2t, b2)


def _l3_merge_kernel(p_ref, u3_ref, o_ref):
    agg = jnp.max(p_ref[...], axis=0, keepdims=True)
    z = u3_ref[...] + agg
    o_ref[...] = jnp.where(agg > NEG * 0.5, jax.nn.sigmoid(z), 0.0)


def _l3_merge(partials, u3row):
    return pl.pallas_call(
        _l3_merge_kernel,
        in_specs=[
            pl.BlockSpec((NW, NPAD), lambda: (0, 0)),
            pl.BlockSpec((1, NPAD), lambda: (0, 0)),
        ],
        out_specs=pl.BlockSpec((1, NPAD), lambda: (0, 0)),
        out_shape=jax.ShapeDtypeStruct((1, NPAD), jnp.float32),
    )(partials, u3row)


@functools.partial(
    pl.kernel,
    out_type=jax.ShapeDtypeStruct((EH, H), jnp.float32),
    mesh=_MESH,
    scratch_types=[
        pltpu.VMEM((EC,), jnp.int32),
        pltpu.VMEM((EC,), jnp.int32),
        pltpu.VMEM((EC, H), jnp.float32),
        pltpu.VMEM((EC, H), jnp.float32),
        pltpu.VMEM((EC,), jnp.int32),
        pltpu.VMEM((EC,), jnp.int32),
        pltpu.VMEM((EC, H), jnp.float32),
        pltpu.VMEM((EC, H), jnp.float32),
        pltpu.SemaphoreType.DMA,
        pltpu.SemaphoreType.DMA,
        pltpu.SemaphoreType.DMA,
        pltpu.SemaphoreType.DMA,
    ],
)
def _h_stage(u_hbm, v_hbm, dst_hbm, src_hbm, h_hbm,
             idx_d0, idx_s0, bu0, bv0, idx_d1, idx_s1, bu1, bv1,
             su0, sv0, su1, sv1):
    w = _wid()
    bufs = ((idx_d0, idx_s0, bu0, bv0, su0, sv0),
            (idx_d1, idx_s1, bu1, bv1, su1, sv1))

    def start(k, slot):
        c = w + k * NW

        @pl.when(c < ECHUNKS)
        def _():
            idx_d, idx_s, bu, bv, su, sv = bufs[slot]
            base = c * EC
            pltpu.sync_copy(dst_hbm.at[pl.ds(base, EC)], idx_d)
            pltpu.sync_copy(src_hbm.at[pl.ds(base, EC)], idx_s)
            pltpu.async_copy(u_hbm.at[idx_d], bu, su)
            pltpu.async_copy(v_hbm.at[idx_s], bv, sv)

    def finish(k, slot):
        c = w + k * NW

        @pl.when(c < ECHUNKS)
        def _():
            idx_d, idx_s, bu, bv, su, sv = bufs[slot]
            base = c * EC
            pltpu.make_async_copy(u_hbm.at[idx_d], bu, su).wait()
            pltpu.make_async_copy(v_hbm.at[idx_s], bv, sv).wait()

            @pl.loop(0, EC, unroll=8)
            def _row(r):
                for j in range(NG):
                    s = pl.ds(j * L, L)
                    bu[r, s] = jnp.maximum(bu[r, s] + bv[r, s], 0.0)

            pltpu.sync_copy(bu, h_hbm.at[pl.ds(base, EC), :])

    start(0, 0)

    @pl.loop(0, EC_PER_W, step=2)
    def _chunk(k):
        start(k + 1, 1)
        finish(k, 0)
        start(k + 2, 0)
        finish(k + 1, 1)


@functools.partial(
    pl.kernel,
    out_type=jax.ShapeDtypeStruct((2, NPAD, H), jnp.float32),
    mesh=_MESH,
    compiler_params=pltpu.CompilerParams(use_tc_tiling_on_sc=False, needs_layout_passes=False),
    scratch_types=[
        pltpu.VMEM((NHALF + L, L), jnp.float32),
        pltpu.VMEM((CH,), jnp.int32),
        pltpu.VMEM((CH,), jnp.int32),
        pltpu.VMEM((CH, L), jnp.float32),
        pltpu.VMEM((CH, L), jnp.float32),
        pltpu.SemaphoreType.DMA,
        pltpu.SemaphoreType.DMA,
        pltpu.SemaphoreType.DMA,
        pltpu.SemaphoreType.DMA,
    ],
)
def _scatter_stage(m_hbm, dst_hbm, out_hbm, table, dbuf0, dbuf1, mbuf0, mbuf1,
                   sd0, sm0, sd1, sm1):
    w = _wid()
    g = w % NG
    h = (w // NG) % 2
    q = w // (NG * 2)
    lo = h * NHALF
    iota = lax.iota(jnp.int32, L)
    bufs = ((dbuf0, mbuf0, sd0, sm0), (dbuf1, mbuf1, sd1, sm1))

    def refs(c, slot):
        base = q * (EH // 2) + c * CH
        dbuf, mbuf, sd, sm = bufs[slot]
        return (dst_hbm.at[pl.ds(base, CH)], dbuf, sd,
                m_hbm.at[pl.ds(base, CH), pl.ds(g * L, L)], mbuf, sm)

    def start(c, slot):
        @pl.when(c < SCHUNKS)
        def _():
            dsrc, dbuf, sd, msrc, mbuf, sm = refs(c, slot)
            pltpu.async_copy(dsrc, dbuf, sd)
            pltpu.async_copy(msrc, mbuf, sm)

    def finish(c, slot):
        @pl.when(c < SCHUNKS)
        def _():
            dsrc, dbuf, sd, msrc, mbuf, sm = refs(c, slot)
            pltpu.make_async_copy(dsrc, dbuf, sd).wait()
            pltpu.make_async_copy(msrc, mbuf, sm).wait()

            @pl.loop(0, CH // L, unroll=4)
            def _grp(t):
                dvec = dbuf[pl.ds(t * L, L)]
                valid = (dvec >= lo) & (dvec < lo + NHALF)
                dloc = jnp.where(valid, dvec - lo, NHALF)
                for i in range(L):
                    dsp = jnp.take(dloc, jnp.full((L,), i, jnp.int32))
                    vals = mbuf[t * L + i, pl.ds(0, L)]
                    cur = plsc.load_gather(table, [dsp, iota])
                    plsc.store_scatter(table, [dsp, iota], jnp.maximum(cur, vals))

    @pl.loop(0, (NHALF + L) // L)
    def _init(r):
        for j in range(L):
            table[r * L + j, pl.ds(0, L)] = jnp.full((L,), NEG, jnp.float32)

    start(0, 0)

    @pl.loop(0, SCHUNKS, step=2)
    def _chunk(c):
        start(c + 1, 1)
        finish(c, 0)
        start(c + 2, 0)
        finish(c + 1, 1)

    pltpu.sync_copy(table.at[pl.ds(0, NHALF), :],
                    out_hbm.at[q, pl.ds(lo, NHALF), pl.ds(g * L, L)])


@functools.partial(
    pl.kernel,
    out_type=jax.ShapeDtypeStruct((NW, NPAD), jnp.float32),
    mesh=_MESH,
    compiler_params=pltpu.CompilerParams(use_tc_tiling_on_sc=False, needs_layout_passes=False),
    scratch_types=[
        pltpu.VMEM((NPAD,), jnp.float32),
        pltpu.VMEM((NPAD + L,), jnp.float32),
        pltpu.VMEM((CH3,), jnp.int32),
        pltpu.VMEM((CH3,), jnp.int32),
    ],
)
def _layer3_stage(v3_hbm, dst_hbm, src_hbm, out_hbm, v3buf, table, dbuf, sbuf):
    w = _wid()
    pltpu.sync_copy(v3_hbm, v3buf)

    @pl.loop(0, (NPAD + L) // L)
    def _init(r):
        table[pl.ds(r * L, L)] = jnp.full((L,), NEG, jnp.float32)

    @pl.loop(0, DCHUNKS3 // NW)
    def _chunk(k):
        c = w + k * NW
        base = c * CH3
        pltpu.sync_copy(dst_hbm.at[pl.ds(base, CH3)], dbuf)
        pltpu.sync_copy(src_hbm.at[pl.ds(base, CH3)], sbuf)

        @pl.loop(0, CH3 // L)
        def _grp(t):
            dvec = dbuf[pl.ds(t * L, L)]
            svec = sbuf[pl.ds(t * L, L)]
            vals = plsc.load_gather(v3buf, [svec])
            for i in range(L):
                dsp = jnp.take(dvec, jnp.full((L,), i, jnp.int32))
                vsp = jnp.take(vals, jnp.full((L,), i, jnp.int32))
                cur = plsc.load_gather(table, [dsp])
                plsc.store_scatter(table, [dsp], jnp.maximum(cur, vsp))

    pltpu.sync_copy(table.at[pl.ds(0, NPAD)], out_hbm.at[w, :])


def kernel(x, edge_index, edge_attr, W1, b1, W2, b2, W3, b3, W4, b4, W5, b5):
    src = edge_index[0]
    dst = edge_index[1]
    IN = x.shape[1]

    def split(Wa):
        return (Wa[:, :IN] - Wa[:, IN:]).T, Wa[:, IN:].T

    dstA, dstB = dst[:EH], dst[EH:]
    srcA, srcB = src[:EH], src[EH:]

    def layer_edges(u, v, Wb, bb):
        hA = _h_stage(u, v, dstA, srcA)
        hB = _h_stage(u, v, dstB, srcB)
        mA = _edge_mm(hA, Wb.T, bb[None, :])
        mB = _edge_mm(hB, Wb.T, bb[None, :])
        pA = _scatter_stage(mA, dstA)
        pB = _scatter_stage(mB, dstB)
        return pA, pB

    wd1, ws1 = split(W1)
    u1, v1 = _node_uv(x, wd1, ws1, b1[None, :])
    pa1, pb1 = layer_edges(u1, v1, W2, b2)

    wd2, ws2 = split(W3)
    u2, v2 = _node_uv_blocked(pa1, pb1, wd2, ws2, b3[None, :])
    pa2, pb2 = layer_edges(u2, v2, W4, b4)

    a = W5[0, :H]
    bcol = W5[0, H:]
    wcat = jnp.zeros((H, H), jnp.float32).at[:, 0].set(a - bcol).at[:, 1].set(bcol)
    brow = jnp.zeros((1, H), jnp.float32).at[0, 0].set(b5[0])
    uv3, _ = _node_uv_blocked(pa2, pb2, wcat, wcat, brow)
    u3 = uv3[:, 0]
    v3 = uv3[:, 1]
    p3 = _layer3_stage(v3, dst, src)
    out = _l3_merge(p3, u3[None, :])
    return out[0, :N, None]


# speedup vs baseline: 1.0025x; 1.0025x over previous
import functools

import jax
import jax.numpy as jnp
from jax import lax
from jax.experimental import pallas as pl
from jax.experimental.pallas import tpu as pltpu
from jax.experimental.pallas import tpu_sc as plsc

N = 10000
E = 320000
H = 128

NC = 2
NS = 16
NW = NC * NS
L = 16
NG = H // L

NPAD = 10240
NHALF = NPAD // 2

EH = E // 2

EC = 128
ECHUNKS = EH // EC
EC_PER_W = -(-ECHUNKS // NW)

CH = 800
SCHUNKS = (EH // 2) // CH

CH3 = 2000
DCHUNKS3 = E // CH3

NODE_BLK = 2000
NODE_BLK2 = 2048
EDGE_BLK = 2000

_MESH = plsc.VectorSubcoreMesh(core_axis_name="c", subcore_axis_name="s")
NEG = -3.0e38


def _wid():
    return lax.axis_index("s") * NC + lax.axis_index("c")


def _uv_kernel(x_ref, wd_ref, ws_ref, b_ref, u_ref, v_ref):
    x = x_ref[...]
    u_ref[...] = jnp.dot(x, wd_ref[...], preferred_element_type=jnp.float32) + b_ref[...]
    v_ref[...] = jnp.dot(x, ws_ref[...], preferred_element_type=jnp.float32)


def _node_uv(x, wd, ws, b):
    n = x.shape[0]
    blk = NODE_BLK if n == N else NODE_BLK2
    return pl.pallas_call(
        _uv_kernel,
        grid=(n // blk,),
        in_specs=[
            pl.BlockSpec((blk, x.shape[1]), lambda i: (i, 0)),
            pl.BlockSpec((x.shape[1], H), lambda i: (0, 0)),
            pl.BlockSpec((x.shape[1], H), lambda i: (0, 0)),
            pl.BlockSpec((1, H), lambda i: (0, 0)),
        ],
        out_specs=[
            pl.BlockSpec((blk, H), lambda i: (i, 0)),
            pl.BlockSpec((blk, H), lambda i: (i, 0)),
        ],
        out_shape=[
            jax.ShapeDtypeStruct((n, H), jnp.float32),
            jax.ShapeDtypeStruct((n, H), jnp.float32),
        ],
    )(x, wd, ws, b)


def _uv_blocked_kernel(pa_ref, pb_ref, wd_ref, ws_ref, b_ref, u_ref, v_ref):
    x = jnp.maximum(jnp.maximum(pa_ref[0], pa_ref[1]),
                    jnp.maximum(pb_ref[0], pb_ref[1]))
    x = jnp.maximum(x, 0.0)
    u_ref[...] = jnp.dot(x, wd_ref[...], preferred_element_type=jnp.float32) + b_ref[...]
    v_ref[...] = jnp.dot(x, ws_ref[...], preferred_element_type=jnp.float32)


def _node_uv_blocked(pa, pb, wd, ws, b):
    blk = NODE_BLK2
    return pl.pallas_call(
        _uv_blocked_kernel,
        grid=(NPAD // blk,),
        in_specs=[
            pl.BlockSpec((2, blk, H), lambda i: (0, i, 0)),
            pl.BlockSpec((2, blk, H), lambda i: (0, i, 0)),
            pl.BlockSpec((H, H), lambda i: (0, 0)),
            pl.BlockSpec((H, H), lambda i: (0, 0)),
            pl.BlockSpec((1, H), lambda i: (0, 0)),
        ],
        out_specs=[
            pl.BlockSpec((blk, H), lambda i: (i, 0)),
            pl.BlockSpec((blk, H), lambda i: (i, 0)),
        ],
        out_shape=[
            jax.ShapeDtypeStruct((NPAD, H), jnp.float32),
            jax.ShapeDtypeStruct((NPAD, H), jnp.float32),
        ],
    )(pa, pb, wd, ws, b)


def _mm_kernel(h_ref, w_ref, b_ref, m_ref):
    m_ref[...] = jnp.dot(h_ref[...], w_ref[...], preferred_element_type=jnp.float32) + b_ref[...]


def _edge_mm(h, w2t, b2):
    return pl.pallas_call(
        _mm_kernel,
        grid=(EH // EDGE_BLK,),
        in_specs=[
            pl.BlockSpec((EDGE_BLK, H), lambda i: (i, 0)),
            pl.BlockSpec((H, H), lambda i: (0, 0)),
            pl.BlockSpec((1, H), lambda i: (0, 0)),
        ],
        out_specs=pl.BlockSpec((EDGE_BLK, H), lambda i: (i, 0)),
        out_shape=jax.ShapeDtypeStruct((EH, H), jnp.float32),
    )(h, w2t, b2)


def _l3_merge_kernel(p_ref, u3_ref, o_ref):
    agg = jnp.max(p_ref[...], axis=0, keepdims=True)
    z = u3_ref[...] + agg
    o_ref[...] = jnp.where(agg > NEG * 0.5, jax.nn.sigmoid(z), 0.0)


def _l3_merge(partials, u3row):
    return pl.pallas_call(
        _l3_merge_kernel,
        in_specs=[
            pl.BlockSpec((NW, NPAD), lambda: (0, 0)),
            pl.BlockSpec((1, NPAD), lambda: (0, 0)),
        ],
        out_specs=pl.BlockSpec((1, NPAD), lambda: (0, 0)),
        out_shape=jax.ShapeDtypeStruct((1, NPAD), jnp.float32),
    )(partials, u3row)


@functools.partial(
    pl.kernel,
    out_type=jax.ShapeDtypeStruct((EH, H), jnp.float32),
    mesh=_MESH,
    scratch_types=[
        pltpu.VMEM((EC,), jnp.int32),
        pltpu.VMEM((EC,), jnp.int32),
        pltpu.VMEM((EC, H), jnp.float32),
        pltpu.VMEM((EC, H), jnp.float32),
        pltpu.VMEM((EC,), jnp.int32),
        pltpu.VMEM((EC,), jnp.int32),
        pltpu.VMEM((EC, H), jnp.float32),
        pltpu.VMEM((EC, H), jnp.float32),
        pltpu.SemaphoreType.DMA,
        pltpu.SemaphoreType.DMA,
        pltpu.SemaphoreType.DMA,
        pltpu.SemaphoreType.DMA,
    ],
)
def _h_stage(u_hbm, v_hbm, dst_hbm, src_hbm, h_hbm,
             idx_d0, idx_s0, bu0, bv0, idx_d1, idx_s1, bu1, bv1,
             su0, sv0, su1, sv1):
    w = _wid()
    bufs = ((idx_d0, idx_s0, bu0, bv0, su0, sv0),
            (idx_d1, idx_s1, bu1, bv1, su1, sv1))

    def start(k, slot):
        c = w + k * NW

        @pl.when(c < ECHUNKS)
        def _():
            idx_d, idx_s, bu, bv, su, sv = bufs[slot]
            base = c * EC
            pltpu.sync_copy(dst_hbm.at[pl.ds(base, EC)], idx_d)
            pltpu.sync_copy(src_hbm.at[pl.ds(base, EC)], idx_s)
            pltpu.async_copy(u_hbm.at[idx_d], bu, su)
            pltpu.async_copy(v_hbm.at[idx_s], bv, sv)

    def finish(k, slot):
        c = w + k * NW

        @pl.when(c < ECHUNKS)
        def _():
            idx_d, idx_s, bu, bv, su, sv = bufs[slot]
            base = c * EC
            pltpu.make_async_copy(u_hbm.at[idx_d], bu, su).wait()
            pltpu.make_async_copy(v_hbm.at[idx_s], bv, sv).wait()

            @pl.loop(0, EC, unroll=4)
            def _row(r):
                for j in range(NG):
                    s = pl.ds(j * L, L)
                    bu[r, s] = jnp.maximum(bu[r, s] + bv[r, s], 0.0)

            pltpu.sync_copy(bu, h_hbm.at[pl.ds(base, EC), :])

    start(0, 0)

    @pl.loop(0, EC_PER_W, step=2)
    def _chunk(k):
        start(k + 1, 1)
        finish(k, 0)
        start(k + 2, 0)
        finish(k + 1, 1)


@functools.partial(
    pl.kernel,
    out_type=jax.ShapeDtypeStruct((2, NPAD, H), jnp.float32),
    mesh=_MESH,
    compiler_params=pltpu.CompilerParams(use_tc_tiling_on_sc=False, needs_layout_passes=False),
    scratch_types=[
        pltpu.VMEM((NHALF + L, L), jnp.float32),
        pltpu.VMEM((CH,), jnp.int32),
        pltpu.VMEM((CH,), jnp.int32),
        pltpu.VMEM((CH, L), jnp.float32),
        pltpu.VMEM((CH, L), jnp.float32),
        pltpu.SemaphoreType.DMA,
        pltpu.SemaphoreType.DMA,
        pltpu.SemaphoreType.DMA,
        pltpu.SemaphoreType.DMA,
    ],
)
def _scatter_stage(m_hbm, dst_hbm, out_hbm, table, dbuf0, dbuf1, mbuf0, mbuf1,
                   sd0, sm0, sd1, sm1):
    w = _wid()
    g = w % NG
    h = (w // NG) % 2
    q = w // (NG * 2)
    lo = h * NHALF
    iota = lax.iota(jnp.int32, L)
    bufs = ((dbuf0, mbuf0, sd0, sm0), (dbuf1, mbuf1, sd1, sm1))

    def refs(c, slot):
        base = q * (EH // 2) + c * CH
        dbuf, mbuf, sd, sm = bufs[slot]
        return (dst_hbm.at[pl.ds(base, CH)], dbuf, sd,
                m_hbm.at[pl.ds(base, CH), pl.ds(g * L, L)], mbuf, sm)

    def start(c, slot):
        @pl.when(c < SCHUNKS)
        def _():
            dsrc, dbuf, sd, msrc, mbuf, sm = refs(c, slot)
            pltpu.async_copy(dsrc, dbuf, sd)
            pltpu.async_copy(msrc, mbuf, sm)

    def finish(c, slot):
        @pl.when(c < SCHUNKS)
        def _():
            dsrc, dbuf, sd, msrc, mbuf, sm = refs(c, slot)
            pltpu.make_async_copy(dsrc, dbuf, sd).wait()
            pltpu.make_async_copy(msrc, mbuf, sm).wait()

            @pl.loop(0, CH // L, unroll=2)
            def _grp(t):
                dvec = dbuf[pl.ds(t * L, L)]
                valid = (dvec >= lo) & (dvec < lo + NHALF)
                dloc = jnp.where(valid, dvec - lo, NHALF)
                for i in range(L):
                    dsp = jnp.take(dloc, jnp.full((L,), i, jnp.int32))
                    vals = mbuf[t * L + i, pl.ds(0, L)]
                    cur = plsc.load_gather(table, [dsp, iota])
                    plsc.store_scatter(table, [dsp, iota], jnp.maximum(cur, vals))

    @pl.loop(0, (NHALF + L) // L)
    def _init(r):
        for j in range(L):
            table[r * L + j, pl.ds(0, L)] = jnp.full((L,), NEG, jnp.float32)

    start(0, 0)

    @pl.loop(0, SCHUNKS, step=2)
    def _chunk(c):
        start(c + 1, 1)
        finish(c, 0)
        start(c + 2, 0)
        finish(c + 1, 1)

    pltpu.sync_copy(table.at[pl.ds(0, NHALF), :],
                    out_hbm.at[q, pl.ds(lo, NHALF), pl.ds(g * L, L)])


@functools.partial(
    pl.kernel,
    out_type=jax.ShapeDtypeStruct((NW, NPAD), jnp.float32),
    mesh=_MESH,
    compiler_params=pltpu.CompilerParams(use_tc_tiling_on_sc=False, needs_layout_passes=False),
    scratch_types=[
        pltpu.VMEM((NPAD,), jnp.float32),
        pltpu.VMEM((NPAD + L,), jnp.float32),
        pltpu.VMEM((CH3,), jnp.int32),
        pltpu.VMEM((CH3,), jnp.int32),
    ],
)
def _layer3_stage(v3_hbm, dst_hbm, src_hbm, out_hbm, v3buf, table, dbuf, sbuf):
    w = _wid()
    pltpu.sync_copy(v3_hbm, v3buf)

    @pl.loop(0, (NPAD + L) // L)
    def _init(r):
        table[pl.ds(r * L, L)] = jnp.full((L,), NEG, jnp.float32)

    @pl.loop(0, DCHUNKS3 // NW)
    def _chunk(k):
        c = w + k * NW
        base = c * CH3
        pltpu.sync_copy(dst_hbm.at[pl.ds(base, CH3)], dbuf)
        pltpu.sync_copy(src_hbm.at[pl.ds(base, CH3)], sbuf)

        @pl.loop(0, CH3 // L)
        def _grp(t):
            dvec = dbuf[pl.ds(t * L, L)]
            svec = sbuf[pl.ds(t * L, L)]
            vals = plsc.load_gather(v3buf, [svec])
            for i in range(L):
                dsp = jnp.take(dvec, jnp.full((L,), i, jnp.int32))
                vsp = jnp.take(vals, jnp.full((L,), i, jnp.int32))
                cur = plsc.load_gather(table, [dsp])
                plsc.store_scatter(table, [dsp], jnp.maximum(cur, vsp))

    pltpu.sync_copy(table.at[pl.ds(0, NPAD)], out_hbm.at[w, :])


def kernel(x, edge_index, edge_attr, W1, b1, W2, b2, W3, b3, W4, b4, W5, b5):
    src = edge_index[0]
    dst = edge_index[1]
    IN = x.shape[1]

    def split(Wa):
        return (Wa[:, :IN] - Wa[:, IN:]).T, Wa[:, IN:].T

    dstA, dstB = dst[:EH], dst[EH:]
    srcA, srcB = src[:EH], src[EH:]

    def layer_edges(u, v, Wb, bb):
        hA = _h_stage(u, v, dstA, srcA)
        hB = _h_stage(u, v, dstB, srcB)
        mA = _edge_mm(hA, Wb.T, bb[None, :])
        mB = _edge_mm(hB, Wb.T, bb[None, :])
        pA = _scatter_stage(mA, dstA)
        pB = _scatter_stage(mB, dstB)
        return pA, pB

    wd1, ws1 = split(W1)
    u1, v1 = _node_uv(x, wd1, ws1, b1[None, :])
    pa1, pb1 = layer_edges(u1, v1, W2, b2)

    wd2, ws2 = split(W3)
    u2, v2 = _node_uv_blocked(pa1, pb1, wd2, ws2, b3[None, :])
    pa2, pb2 = layer_edges(u2, v2, W4, b4)

    a = W5[0, :H]
    bcol = W5[0, H:]
    wcat = jnp.zeros((H, H), jnp.float32).at[:, 0].set(a - bcol).at[:, 1].set(bcol)
    brow = jnp.zeros((1, H), jnp.float32).at[0, 0].set(b5[0])
    uv3, _ = _node_uv_blocked(pa2, pb2, wcat, wcat, brow)
    u3 = uv3[:, 0]
    v3 = uv3[:, 1]
    p3 = _layer3_stage(v3, dst, src)
    out = _l3_merge(p3, u3[None, :])
    return out[0, :N, None]
